# Initial kernel scaffold; baseline (speedup 1.0000x reference)
import jax
import jax.numpy as jnp
from jax.experimental import pallas as pl


def kernel(fixed_label_map, warped_moving_label_map, vector_field):
    raise NotImplementedError("write your pallas kernel here")


# speedup vs baseline: 25.4874x; 25.4874x over previous
import jax
import jax.numpy as jnp
from jax.experimental import pallas as pl
from jax.experimental.pallas import tpu as pltpu

_NUM_CLASSES = 26
_EPS = 1e-05
_D, _H, _W = 160, 192, 224
_P = 2
_BD = 8
_S = _D // (_P * _BD)
_ROWS = 56


def _main_kernel(f_ref, m_ref, vf_ref, vfp_ref, out_ref):
    j = pl.program_id(1)

    @pl.when(j == 0)
    def _():
        out_ref[...] = jnp.zeros_like(out_ref)

    f = f_ref[...]
    m = m_ref[...]
    cat = jnp.concatenate([f, m], axis=0)
    g = jnp.where(f == m, f, _NUM_CLASSES)

    def body(c, carry):
        denom = jnp.sum(jnp.where(cat == c, 1.0, 0.0), axis=(0, 1),
                        keepdims=True)
        inter = jnp.sum(jnp.where(g == c, 1.0, 0.0), axis=(0, 1),
                        keepdims=True)
        out_ref[pl.ds(c - 1, 1)] = out_ref[pl.ds(c - 1, 1)] + denom
        out_ref[pl.ds(24 + c, 1)] = out_ref[pl.ds(24 + c, 1)] + inter
        return carry

    jax.lax.fori_loop(1, _NUM_CLASSES, body, 0)

    v = vf_ref[...]
    vp = vfp_ref[...]
    vc = jnp.concatenate([vp, v], axis=1)
    dz = vc[:, 1:] - vc[:, :-1]
    dy = v[:, :, 1:, :] - v[:, :, :-1, :]
    dx = v[..., 1:] - v[..., :-1]
    sz = jnp.sum(dz * dz, axis=(0, 1, 2), keepdims=True)
    sy = jnp.sum(dy * dy, axis=(0, 1, 2), keepdims=True)
    sx = jnp.sum(dx * dx, axis=(0, 1, 2), keepdims=True)
    sx = jnp.concatenate([sx, jnp.zeros((1, 1, 1, 1), jnp.float32)], axis=3)
    out_ref[pl.ds(50, 1)] = out_ref[pl.ds(50, 1)] + sz.reshape(1, 1, _W)
    out_ref[pl.ds(51, 1)] = out_ref[pl.ds(51, 1)] + sy.reshape(1, 1, _W)
    out_ref[pl.ds(52, 1)] = out_ref[pl.ds(52, 1)] + sx.reshape(1, 1, _W)


def _epilogue_kernel(a_ref, o_ref):
    a = a_ref[...]
    t = a[:_ROWS] + a[_ROWS:]
    d = jnp.sum(t[0:25], axis=2)
    i = jnp.sum(t[25:50], axis=2)
    dice = (2.0 * i + _EPS) / (d + _EPS)
    sim = 1.0 - jnp.sum(dice) / 25.0
    nz = 3.0 * (_D - 1) * _H * _W
    ny = 3.0 * _D * (_H - 1) * _W
    nx = 3.0 * _D * _H * (_W - 1)
    sz = jnp.sum(t[50]) / nz
    sy = jnp.sum(t[51]) / ny
    sx = jnp.sum(t[52]) / nx
    smooth = (sz + sy + sx) / 3.0
    total = sim + smooth
    lane = jax.lax.broadcasted_iota(jnp.int32, (8, 128), 1)
    o_ref[...] = jnp.where(lane == 0, total,
                           jnp.where(lane == 1, sim,
                                     jnp.where(lane == 2, smooth, 0.0)))


def kernel(fixed_label_map, warped_moving_label_map, vector_field):
    f = fixed_label_map.reshape(_D, _H, _W)
    m = warped_moving_label_map.reshape(_D, _H, _W)
    vf = vector_field.reshape(3, _D, _H, _W)

    part = pl.pallas_call(
        _main_kernel,
        grid=(_P, _S),
        in_specs=[
            pl.BlockSpec((_BD, _H, _W), lambda p, j: (p * _S + j, 0, 0)),
            pl.BlockSpec((_BD, _H, _W), lambda p, j: (p * _S + j, 0, 0)),
            pl.BlockSpec((3, _BD, _H, _W), lambda p, j: (0, p * _S + j, 0, 0)),
            pl.BlockSpec(
                (3, 1, _H, _W),
                lambda p, j: (0, jnp.maximum((p * _S + j) * _BD - 1, 0), 0, 0)),
        ],
        out_specs=pl.BlockSpec((_ROWS, 1, _W), lambda p, j: (p, 0, 0)),
        out_shape=jax.ShapeDtypeStruct((_P * _ROWS, 1, _W), jnp.float32),
        compiler_params=pltpu.CompilerParams(
            dimension_semantics=("parallel", "arbitrary"),
            vmem_limit_bytes=48 * 1024 * 1024,
        ),
    )(f, m, vf, vf)

    res = pl.pallas_call(
        _epilogue_kernel,
        in_specs=[pl.BlockSpec((_P * _ROWS, 1, _W), lambda: (0, 0, 0))],
        out_specs=pl.BlockSpec((8, 128), lambda: (0, 0)),
        out_shape=jax.ShapeDtypeStruct((8, 128), jnp.float32),
    )(part)

    return (res[0, 0], res[0, 1], res[0, 2])
